# Initial kernel scaffold; baseline (speedup 1.0000x reference)
import jax
import jax.numpy as jnp
from jax.experimental import pallas as pl


def kernel(x, edge_index, W_self, W_neigh, ln_gamma, ln_beta):
    raise NotImplementedError("write your pallas kernel here")


# speedup vs baseline: 2.6403x; 2.6403x over previous
import functools

import jax
import jax.numpy as jnp
from jax import lax
from jax.experimental import pallas as pl
import jax.experimental.pallas.tpu as pltpu
from jax.experimental.pallas import tpu_sc as plsc

N = 10000
E = 320000
D = 128
NP = 10240
CHUNK = 64
NCHUNKS_PAD = 5024
CHUNKS_PER_TILE = NCHUNKS_PAD // 16
EPAD = NCHUNKS_PAD * CHUNK
TROWS = NP // 16
BLK = 64


def _sc_aggregate(x, srcp, dstp, zf, ones):
  mesh = plsc.VectorSubcoreMesh(core_axis_name="c", subcore_axis_name="s")

  @functools.partial(
      pl.kernel,
      out_type=jax.ShapeDtypeStruct((2, NP, D), jnp.float32),
      mesh=mesh,
      scratch_types=[
          pltpu.VMEM_SHARED((NP, D), jnp.float32),
          pltpu.VMEM((CHUNK,), jnp.int32),
          pltpu.VMEM((CHUNK,), jnp.int32),
          pltpu.VMEM((CHUNK, D), jnp.float32),
          pltpu.SemaphoreType.DMA,
      ],
  )
  def body(x_hbm, src_hbm, dst_hbm, zf_hbm, ones_hbm, out_hbm,
           acc_sh, src_v, dst_v, rows_v, sem):
    cid = lax.axis_index("c")
    sid = lax.axis_index("s")

    base = sid * TROWS
    pltpu.sync_copy(zf_hbm, rows_v)
    for k in range(TROWS // BLK):
      pltpu.sync_copy(rows_v, acc_sh.at[pl.ds(base + k * BLK, BLK)])
    plsc.subcore_barrier()

    cbase = sid * CHUNKS_PER_TILE

    @pl.when(cid == 0)
    def _features():
      def step(k, carry):
        ebase = (cbase + k) * CHUNK
        pltpu.sync_copy(src_hbm.at[pl.ds(ebase, CHUNK)], src_v)
        pltpu.async_copy(x_hbm.at[src_v], rows_v, sem).wait()
        pltpu.sync_copy(dst_hbm.at[pl.ds(ebase, CHUNK)], dst_v)
        pltpu.sync_copy(rows_v, acc_sh.at[dst_v], add=True)
        return carry

      lax.fori_loop(0, CHUNKS_PER_TILE, step, 0)

    @pl.when(cid == 1)
    def _counts():
      pltpu.sync_copy(ones_hbm, rows_v)

      def step(k, carry):
        ebase = (cbase + k) * CHUNK
        pltpu.sync_copy(dst_hbm.at[pl.ds(ebase, CHUNK)], dst_v)
        pltpu.sync_copy(rows_v, acc_sh.at[dst_v], add=True)
        return carry

      lax.fori_loop(0, CHUNKS_PER_TILE, step, 0)

    plsc.subcore_barrier()

    for k in range(TROWS // BLK):
      off = base + k * BLK
      pltpu.sync_copy(acc_sh.at[pl.ds(off, BLK)], rows_v)
      pltpu.sync_copy(rows_v, out_hbm.at[cid, pl.ds(off, BLK)])

  return body(x, srcp, dstp, zf, ones)


def _tc_body(x_ref, agg_ref, cnt_ref, wst_ref, wnt_ref, g_ref, b_ref,
             out_ref):
  cnt = cnt_ref[:, 0:1]
  mean = agg_ref[...] / (cnt + 1e-9)
  h = (jnp.dot(x_ref[...], wst_ref[...], preferred_element_type=jnp.float32)
       + jnp.dot(mean, wnt_ref[...], preferred_element_type=jnp.float32))
  mu = jnp.mean(h, axis=-1, keepdims=True)
  var = jnp.mean((h - mu) ** 2, axis=-1, keepdims=True)
  out_ref[...] = (h - mu) / jnp.sqrt(var + 1e-5) * g_ref[...] + b_ref[...]


def _tc_finish(x, agg, cnt, wst, wnt, gamma, beta):
  BR = 1024
  return pl.pallas_call(
      _tc_body,
      grid=(10,),
      in_specs=[
          pl.BlockSpec((BR, D), lambda i: (i, 0)),
          pl.BlockSpec((BR, D), lambda i: (i, 0)),
          pl.BlockSpec((BR, D), lambda i: (i, 0)),
          pl.BlockSpec((D, D), lambda i: (0, 0)),
          pl.BlockSpec((D, D), lambda i: (0, 0)),
          pl.BlockSpec((1, D), lambda i: (0, 0)),
          pl.BlockSpec((1, D), lambda i: (0, 0)),
      ],
      out_specs=pl.BlockSpec((BR, D), lambda i: (i, 0)),
      out_shape=jax.ShapeDtypeStruct((N, D), jnp.float32),
  )(x, agg, cnt, wst, wnt, gamma, beta)


def kernel(x, edge_index, W_self, W_neigh, ln_gamma, ln_beta):
  ei = edge_index.astype(jnp.int32)
  pad = EPAD - E
  srcp = jnp.concatenate([ei[0], jnp.zeros((pad,), jnp.int32)])
  dstp = jnp.concatenate([ei[1], jnp.full((pad,), N, jnp.int32)])
  zf = jnp.zeros((BLK, D), jnp.float32)
  ones = jnp.ones((CHUNK, D), jnp.float32)
  out = _sc_aggregate(x, srcp, dstp, zf, ones)
  return _tc_finish(x, out[0], out[1],
                    W_self.T, W_neigh.T,
                    ln_gamma.reshape(1, D), ln_beta.reshape(1, D))


# speedup vs baseline: 4.5942x; 1.7400x over previous
import functools

import jax
import jax.numpy as jnp
from jax import lax
from jax.experimental import pallas as pl
import jax.experimental.pallas.tpu as pltpu
from jax.experimental.pallas import tpu_sc as plsc

N = 10000
E = 320000
D = 128
NP = 10240
CHUNK = 128
CPT = 158
NCHUNKS_IDX = 16 * CPT + 2
EPADI = NCHUNKS_IDX * CHUNK
TROWS = NP // 16
BLK = 64


def _sc_aggregate(x, srcp, dstp, zf, ones):
  mesh = plsc.VectorSubcoreMesh(core_axis_name="c", subcore_axis_name="s")

  @functools.partial(
      pl.kernel,
      out_type=jax.ShapeDtypeStruct((2, NP, D), jnp.float32),
      mesh=mesh,
      scratch_types=[
          pltpu.VMEM_SHARED((NP, D), jnp.float32),
          pltpu.VMEM((CHUNK, D), jnp.float32),
          pltpu.VMEM((CHUNK, D), jnp.float32),
          pltpu.VMEM((CHUNK,), jnp.int32),
          pltpu.VMEM((CHUNK,), jnp.int32),
          pltpu.VMEM((CHUNK,), jnp.int32),
          pltpu.VMEM((CHUNK,), jnp.int32),
          pltpu.VMEM((CHUNK,), jnp.int32),
          pltpu.VMEM((CHUNK,), jnp.int32),
          pltpu.VMEM((CHUNK,), jnp.int32),
          pltpu.SemaphoreType.DMA,
          pltpu.SemaphoreType.DMA,
          pltpu.SemaphoreType.DMA,
          pltpu.SemaphoreType.DMA,
          pltpu.SemaphoreType.DMA,
          pltpu.SemaphoreType.DMA,
          pltpu.SemaphoreType.DMA,
          pltpu.SemaphoreType.DMA,
          pltpu.SemaphoreType.DMA,
          pltpu.SemaphoreType.DMA,
          pltpu.SemaphoreType.DMA,
      ],
  )
  def body(x_hbm, src_hbm, dst_hbm, zf_hbm, ones_hbm, out_hbm,
           acc_sh, rows0, rows1, si0, si1, si2, di0, di1, di2, di3,
           semg0, semg1, sems0, sems1,
           semis0, semis1, semis2, semid0, semid1, semid2, semid3):
    cid = lax.axis_index("c")
    sid = lax.axis_index("s")
    rows = (rows0, rows1)
    si = (si0, si1, si2)
    di = (di0, di1, di2, di3)
    semg = (semg0, semg1)
    sems = (sems0, sems1)
    semis = (semis0, semis1, semis2)
    semid = (semid0, semid1, semid2, semid3)

    base = sid * TROWS
    pltpu.sync_copy(zf_hbm, rows0.at[pl.ds(0, BLK)])
    for k in range(TROWS // BLK):
      pltpu.sync_copy(rows0.at[pl.ds(0, BLK)],
                      acc_sh.at[pl.ds(base + k * BLK, BLK)])
    plsc.subcore_barrier()

    cbase = sid * CPT

    def ld_src(c, j, sem=None):
      sl = src_hbm.at[pl.ds((cbase + c) * CHUNK, CHUNK)]
      if sem is None:
        pltpu.sync_copy(sl, si[j])
      else:
        pltpu.async_copy(sl, si[j], sem)

    def ld_dst(c, j, sem=None):
      sl = dst_hbm.at[pl.ds((cbase + c) * CHUNK, CHUNK)]
      if sem is None:
        pltpu.sync_copy(sl, di[j])
      else:
        pltpu.async_copy(sl, di[j], sem)

    @pl.when(cid == 0)
    def _features():
      ld_src(0, 0)
      ld_dst(0, 0)
      ld_src(1, 1)
      ld_dst(1, 1)
      g0 = pltpu.async_copy(x_hbm.at[si[0]], rows[0], semg[0])
      g1 = pltpu.async_copy(x_hbm.at[si[1]], rows[1], semg[1])
      ld_src(2, 2, semis[2])
      ld_dst(2, 2, semid[2])
      g0.wait()
      s0 = pltpu.async_copy(rows[0], acc_sh.at[di[0]], sems[0], add=True)
      pltpu.make_async_copy(src_hbm.at[pl.ds(0, CHUNK)], si[2], semis[2]).wait()
      pltpu.make_async_copy(dst_hbm.at[pl.ds(0, CHUNK)], di[2], semid[2]).wait()
      s0.wait()
      pltpu.async_copy(x_hbm.at[si[2]], rows[0], semg[0])
      g1.wait()
      pltpu.async_copy(rows[1], acc_sh.at[di[1]], sems[1], add=True)
      ld_src(3, 0, semis[0])
      ld_dst(3, 0, semid[0])

      def group(K, carry):
        for i in range(6):
          k = 2 + i
          j = k % 3
          j1 = (k + 1) % 3
          j2 = (k + 2) % 3
          b = k % 2
          b1 = 1 - b
          c = 6 * K + 2 + i
          pltpu.make_async_copy(
              src_hbm.at[pl.ds(0, CHUNK)], si[j1], semis[j1]).wait()
          pltpu.make_async_copy(
              dst_hbm.at[pl.ds(0, CHUNK)], di[j1], semid[j1]).wait()
          pltpu.make_async_copy(
              rows[b1], acc_sh.at[pl.ds(0, CHUNK)], sems[b1]).wait()
          pltpu.async_copy(x_hbm.at[si[j1]], rows[b1], semg[b1])
          pltpu.make_async_copy(
              x_hbm.at[pl.ds(0, CHUNK)], rows[b], semg[b]).wait()
          pltpu.async_copy(rows[b], acc_sh.at[di[j]], sems[b], add=True)
          ld_src(c + 2, j2, semis[j2])
          ld_dst(c + 2, j2, semid[j2])
        return carry

      lax.fori_loop(0, (CPT - 2) // 6, group, 0)

      pltpu.make_async_copy(
          rows[1], acc_sh.at[pl.ds(0, CHUNK)], sems[1]).wait()
      pltpu.make_async_copy(
          x_hbm.at[pl.ds(0, CHUNK)], rows[0], semg[0]).wait()
      pltpu.make_async_copy(
          src_hbm.at[pl.ds(0, CHUNK)], si[0], semis[0]).wait()
      pltpu.make_async_copy(
          dst_hbm.at[pl.ds(0, CHUNK)], di[0], semid[0]).wait()

    @pl.when(cid == 1)
    def _counts():
      pltpu.sync_copy(ones_hbm, rows[0])
      ld_dst(0, 0)
      ld_dst(1, 1)
      pltpu.async_copy(rows[0], acc_sh.at[di[0]], sems[0], add=True)
      ld_dst(2, 2, semid[2])
      pltpu.async_copy(rows[0], acc_sh.at[di[1]], sems[1], add=True)
      ld_dst(3, 3, semid[3])

      def group(K, carry):
        for i in range(4):
          k = 2 + i
          j = k % 4
          j2 = (k + 2) % 4
          b = k % 2
          c = 4 * K + 2 + i
          pltpu.make_async_copy(
              dst_hbm.at[pl.ds(0, CHUNK)], di[j], semid[j]).wait()
          pltpu.make_async_copy(
              rows[0], acc_sh.at[pl.ds(0, CHUNK)], sems[b]).wait()
          pltpu.async_copy(rows[0], acc_sh.at[di[j]], sems[b], add=True)
          ld_dst(c + 2, j2, semid[j2])
        return carry

      lax.fori_loop(0, (CPT - 2) // 4, group, 0)

      pltpu.make_async_copy(
          rows[0], acc_sh.at[pl.ds(0, CHUNK)], sems[0]).wait()
      pltpu.make_async_copy(
          rows[0], acc_sh.at[pl.ds(0, CHUNK)], sems[1]).wait()
      pltpu.make_async_copy(
          dst_hbm.at[pl.ds(0, CHUNK)], di[2], semid[2]).wait()
      pltpu.make_async_copy(
          dst_hbm.at[pl.ds(0, CHUNK)], di[3], semid[3]).wait()

    plsc.subcore_barrier()

    for k in range(TROWS // BLK):
      off = base + k * BLK
      pltpu.sync_copy(acc_sh.at[pl.ds(off, BLK)], rows0.at[pl.ds(0, BLK)])
      pltpu.sync_copy(rows0.at[pl.ds(0, BLK)], out_hbm.at[cid, pl.ds(off, BLK)])

  return body(x, srcp, dstp, zf, ones)


def _tc_body(x_ref, agg_ref, cnt_ref, wst_ref, wnt_ref, g_ref, b_ref,
             out_ref):
  cnt = cnt_ref[:, 0:1]
  mean = agg_ref[...] / (cnt + 1e-9)
  h = (jnp.dot(x_ref[...], wst_ref[...], preferred_element_type=jnp.float32)
       + jnp.dot(mean, wnt_ref[...], preferred_element_type=jnp.float32))
  mu = jnp.mean(h, axis=-1, keepdims=True)
  var = jnp.mean((h - mu) ** 2, axis=-1, keepdims=True)
  out_ref[...] = (h - mu) / jnp.sqrt(var + 1e-5) * g_ref[...] + b_ref[...]


def _tc_finish(x, agg, cnt, wst, wnt, gamma, beta):
  BR = 1024
  return pl.pallas_call(
      _tc_body,
      grid=(10,),
      in_specs=[
          pl.BlockSpec((BR, D), lambda i: (i, 0)),
          pl.BlockSpec((BR, D), lambda i: (i, 0)),
          pl.BlockSpec((BR, D), lambda i: (i, 0)),
          pl.BlockSpec((D, D), lambda i: (0, 0)),
          pl.BlockSpec((D, D), lambda i: (0, 0)),
          pl.BlockSpec((1, D), lambda i: (0, 0)),
          pl.BlockSpec((1, D), lambda i: (0, 0)),
      ],
      out_specs=pl.BlockSpec((BR, D), lambda i: (i, 0)),
      out_shape=jax.ShapeDtypeStruct((N, D), jnp.float32),
  )(x, agg, cnt, wst, wnt, gamma, beta)


def kernel(x, edge_index, W_self, W_neigh, ln_gamma, ln_beta):
  ei = edge_index.astype(jnp.int32)
  pad = EPADI - E
  srcp = jnp.concatenate([ei[0], jnp.zeros((pad,), jnp.int32)])
  dstp = jnp.concatenate([ei[1], jnp.full((pad,), N, jnp.int32)])
  zf = jnp.zeros((BLK, D), jnp.float32)
  ones = jnp.ones((CHUNK, D), jnp.float32)
  out = _sc_aggregate(x, srcp, dstp, zf, ones)
  return _tc_finish(x, out[0], out[1],
                    W_self.T, W_neigh.T,
                    ln_gamma.reshape(1, D), ln_beta.reshape(1, D))


# speedup vs baseline: 6.6019x; 1.4370x over previous
import functools

import jax
import jax.numpy as jnp
from jax import lax
from jax.experimental import pallas as pl
import jax.experimental.pallas.tpu as pltpu
from jax.experimental.pallas import tpu_sc as plsc

N = 10000
E = 320000
D = 128
NP = 10240
C0 = 64
CPT0 = 314
C1 = 128
CPT1 = 158
EPADI = 323840
TROWS = NP // 16
BLK = 64


def _sc_aggregate(x, srcp, dstp, zf, ones):
  mesh = plsc.VectorSubcoreMesh(core_axis_name="c", subcore_axis_name="s")

  @functools.partial(
      pl.kernel,
      out_type=jax.ShapeDtypeStruct((2, NP, D), jnp.float32),
      mesh=mesh,
      scratch_types=[
          pltpu.VMEM_SHARED((NP, D), jnp.float32),
          pltpu.VMEM((4 * C0, D), jnp.float32),
          pltpu.VMEM((4, C0), jnp.int32),
          pltpu.VMEM((6, C0), jnp.int32),
          pltpu.VMEM((C1,), jnp.int32),
          pltpu.VMEM((C1,), jnp.int32),
          pltpu.VMEM((C1,), jnp.int32),
          pltpu.VMEM((C1,), jnp.int32),
          pltpu.SemaphoreType.DMA,
          pltpu.SemaphoreType.DMA,
          pltpu.SemaphoreType.DMA,
          pltpu.SemaphoreType.DMA,
          pltpu.SemaphoreType.DMA,
          pltpu.SemaphoreType.DMA,
          pltpu.SemaphoreType.DMA,
          pltpu.SemaphoreType.DMA,
          pltpu.SemaphoreType.DMA,
          pltpu.SemaphoreType.DMA,
          pltpu.SemaphoreType.DMA,
          pltpu.SemaphoreType.DMA,
          pltpu.SemaphoreType.DMA,
          pltpu.SemaphoreType.DMA,
          pltpu.SemaphoreType.DMA,
          pltpu.SemaphoreType.DMA,
      ],
  )
  def body(x_hbm, src_hbm, dst_hbm, zf_hbm, ones_hbm, out_hbm,
           acc_sh, rowsb, sib, dib, e0, e1, e2, e3,
           semg0, semg1, semg2, semg3, sems0, sems1,
           semis0, semis1, semis2, semis3,
           semid0, semid1, semid2, semid3, semid4, semid5):
    cid = lax.axis_index("c")
    sid = lax.axis_index("s")
    semg = (semg0, semg1, semg2, semg3)
    sems = (sems0, sems1)
    semis = (semis0, semis1, semis2, semis3)
    semid = (semid0, semid1, semid2, semid3, semid4, semid5)
    e = (e0, e1, e2, e3)

    def rows(r):
      return rowsb.at[pl.ds(C0 * r, C0)]

    base = sid * TROWS
    pltpu.sync_copy(zf_hbm, rows(0))
    for k in range(TROWS // BLK):
      pltpu.sync_copy(rows(0), acc_sh.at[pl.ds(base + k * BLK, BLK)])
    plsc.subcore_barrier()

    @pl.when(cid == 0)
    def _features_main():
      cbase = sid * CPT0

      def ld_src(c, sem=None):
        sl = src_hbm.at[pl.ds((cbase + c) * C0, C0)]
        if sem is None:
          pltpu.sync_copy(sl, sib.at[c % 4])
        else:
          pltpu.async_copy(sl, sib.at[c % 4], sem)

      def ld_dst(c, sem=None):
        sl = dst_hbm.at[pl.ds((cbase + c) * C0, C0)]
        if sem is None:
          pltpu.sync_copy(sl, dib.at[c % 6])
        else:
          pltpu.async_copy(sl, dib.at[c % 6], sem)

      def gather(c):
        pltpu.async_copy(x_hbm.at[sib.at[c % 4]], rows(c % 4), semg[c % 4])

      def scatter(c):
        pltpu.async_copy(rows(c % 4), acc_sh.at[dib.at[c % 6]],
                         sems[c % 2], add=True)

      def wait_i(c):
        pltpu.make_async_copy(src_hbm.at[pl.ds(0, C0)], sib.at[c % 4],
                              semis[c % 4]).wait()

      def wait_d(c):
        pltpu.make_async_copy(dst_hbm.at[pl.ds(0, C0)], dib.at[c % 6],
                              semid[c % 6]).wait()

      def wait_g(c):
        pltpu.make_async_copy(x_hbm.at[pl.ds(0, C0)], rows(c % 4),
                              semg[c % 4]).wait()

      def wait_s(c):
        pltpu.make_async_copy(rows(0), acc_sh.at[pl.ds(0, C0)],
                              sems[c % 2]).wait()

      ld_src(0)
      ld_dst(0)
      ld_src(1)
      ld_dst(1)
      gather(0)
      gather(1)
      ld_src(2, semis[2])
      ld_dst(2, semid[2])
      wait_i(2)
      gather(2)
      ld_src(3, semis[3])
      ld_dst(3, semid[3])
      wait_g(0)
      scatter(0)
      wait_i(3)
      gather(3)
      ld_src(4, semis[0])
      ld_dst(4, semid[4])
      wait_g(1)
      scatter(1)

      def group(K, carry):
        for i in range(12):
          c = 12 * K + 2 + i
          ph = 2 + i
          del ph
          wait_s(i)
          wait_i(i + 4)
          pltpu.async_copy(x_hbm.at[sib.at[(i + 4) % 4]],
                           rows((i + 4) % 4), semg[(i + 4) % 4])
          pltpu.async_copy(
              src_hbm.at[pl.ds((cbase + c + 3) * C0, C0)],
              sib.at[(i + 5) % 4], semis[(i + 5) % 4])
          pltpu.async_copy(
              dst_hbm.at[pl.ds((cbase + c + 3) * C0, C0)],
              dib.at[(i + 5) % 6], semid[(i + 5) % 6])
          wait_d(i + 2)
          wait_g(i + 2)
          pltpu.async_copy(rows((i + 2) % 4),
                           acc_sh.at[dib.at[(i + 2) % 6]],
                           sems[i % 2], add=True)
        return carry

      lax.fori_loop(0, (CPT0 - 2) // 12, group, 0)

      wait_s(0)
      wait_s(1)
      wait_g(314)
      wait_g(315)
      wait_i(316)
      wait_d(314)
      wait_d(315)
      wait_d(316)

    @pl.when(cid == 1)
    def _counts():
      cbase = sid * CPT1
      ones_v = rowsb.at[pl.ds(0, C1)]
      ev = (e0, e1, e2, e3)
      semid4 = (semid0, semid1, semid2, semid3)
      pltpu.sync_copy(ones_hbm, ones_v)

      def ld(c, sem=None):
        sl = dst_hbm.at[pl.ds((cbase + c) * C1, C1)]
        if sem is None:
          pltpu.sync_copy(sl, ev[c % 4])
        else:
          pltpu.async_copy(sl, ev[c % 4], sem)

      def wait_e(c):
        pltpu.make_async_copy(dst_hbm.at[pl.ds(0, C1)], ev[c % 4],
                              semid4[c % 4]).wait()

      def wait_sc(c):
        pltpu.make_async_copy(ones_v, acc_sh.at[pl.ds(0, C1)],
                              sems[c % 2]).wait()

      def scatter(c):
        pltpu.async_copy(ones_v, acc_sh.at[ev[c % 4]], sems[c % 2], add=True)

      ld(0)
      ld(1)
      scatter(0)
      ld(2, semid4[2])
      scatter(1)
      ld(3, semid4[3])

      def group(K, carry):
        for i in range(4):
          c = 4 * K + 2 + i
          wait_e(i + 2)
          wait_sc(i)
          pltpu.async_copy(ones_v, acc_sh.at[ev[(i + 2) % 4]],
                           sems[i % 2], add=True)
          pltpu.async_copy(
              dst_hbm.at[pl.ds((cbase + c + 2) * C1, C1)],
              ev[i % 4], semid4[i % 4])
        return carry

      lax.fori_loop(0, (CPT1 - 2) // 4, group, 0)

      wait_sc(0)
      wait_sc(1)
      wait_e(2)
      wait_e(3)

    plsc.subcore_barrier()

    for k in range(TROWS // BLK):
      off = base + k * BLK
      pltpu.sync_copy(acc_sh.at[pl.ds(off, BLK)], rows(0))
      pltpu.sync_copy(rows(0), out_hbm.at[cid, pl.ds(off, BLK)])

  return body(x, srcp, dstp, zf, ones)


def _tc_body(x_ref, agg_ref, cnt_ref, wst_ref, wnt_ref, g_ref, b_ref,
             out_ref):
  cnt = cnt_ref[:, 0:1]
  mean = agg_ref[...] / (cnt + 1e-9)
  h = (jnp.dot(x_ref[...], wst_ref[...], preferred_element_type=jnp.float32)
       + jnp.dot(mean, wnt_ref[...], preferred_element_type=jnp.float32))
  mu = jnp.mean(h, axis=-1, keepdims=True)
  var = jnp.mean((h - mu) ** 2, axis=-1, keepdims=True)
  out_ref[...] = (h - mu) / jnp.sqrt(var + 1e-5) * g_ref[...] + b_ref[...]


def _tc_finish(x, agg, cnt, wst, wnt, gamma, beta):
  BR = 1024
  return pl.pallas_call(
      _tc_body,
      grid=(10,),
      in_specs=[
          pl.BlockSpec((BR, D), lambda i: (i, 0)),
          pl.BlockSpec((BR, D), lambda i: (i, 0)),
          pl.BlockSpec((BR, D), lambda i: (i, 0)),
          pl.BlockSpec((D, D), lambda i: (0, 0)),
          pl.BlockSpec((D, D), lambda i: (0, 0)),
          pl.BlockSpec((1, D), lambda i: (0, 0)),
          pl.BlockSpec((1, D), lambda i: (0, 0)),
      ],
      out_specs=pl.BlockSpec((BR, D), lambda i: (i, 0)),
      out_shape=jax.ShapeDtypeStruct((N, D), jnp.float32),
  )(x, agg, cnt, wst, wnt, gamma, beta)


def kernel(x, edge_index, W_self, W_neigh, ln_gamma, ln_beta):
  ei = edge_index.astype(jnp.int32)
  pad = EPADI - E
  srcp = jnp.concatenate([ei[0], jnp.zeros((pad,), jnp.int32)])
  dstp = jnp.concatenate([ei[1], jnp.full((pad,), N, jnp.int32)])
  zf = jnp.zeros((BLK, D), jnp.float32)
  ones = jnp.ones((C1, D), jnp.float32)
  out = _sc_aggregate(x, srcp, dstp, zf, ones)
  return _tc_finish(x, out[0], out[1],
                    W_self.T, W_neigh.T,
                    ln_gamma.reshape(1, D), ln_beta.reshape(1, D))
